# Initial kernel scaffold; baseline (speedup 1.0000x reference)
import jax
import jax.numpy as jnp
from jax.experimental import pallas as pl


def kernel(g1, h, section_feature):
    raise NotImplementedError("write your pallas kernel here")


# speedup vs baseline: 1.1751x; 1.1751x over previous
import functools

import jax
import jax.numpy as jnp
from jax import lax
from jax.experimental import pallas as pl
from jax.experimental.pallas import tpu as pltpu
from jax.experimental.pallas import tpu_sc as plsc

N = 2048
D = 128
B = 4
K = 512
CH = 256


def _score_topk_kernel(h_ref, sf_ref, hs_ref, gidx_ref):
    b = pl.program_id(0)
    h = h_ref[0]
    sf = sf_ref[0]
    w_col = lax.dot_general(
        h.astype(jnp.bfloat16), sf.astype(jnp.bfloat16),
        (((1,), (0,)), ((), ())),
        preferred_element_type=jnp.float32)
    s_col = jax.nn.sigmoid(w_col)
    hs_ref[0] = h * s_col
    s_row = jnp.transpose(s_col)

    jidx = lax.broadcasted_iota(jnp.int32, (CH, N), 1)
    ranks = []
    for c in range(N // CH):
        s_i = lax.slice(s_col, (c * CH, 0), ((c + 1) * CH, 1))
        iidx = c * CH + lax.broadcasted_iota(jnp.int32, (CH, N), 0)
        gt = (s_row > s_i) | ((s_row == s_i) & (jidx < iidx))
        ranks.append(jnp.sum(gt.astype(jnp.float32), axis=1, keepdims=True))
    rank_col = jnp.concatenate(ranks, axis=0)

    r_row = lax.broadcasted_iota(jnp.int32, (1, K), 1).astype(jnp.float32)
    acc = jnp.zeros((1, K), jnp.float32)
    for c in range(N // CH):
        rk = lax.slice(rank_col, (c * CH, 0), ((c + 1) * CH, 1))
        ival = (b * N + c * CH).astype(jnp.float32) + \
            lax.broadcasted_iota(jnp.int32, (CH, 1), 0).astype(jnp.float32)
        acc = acc + jnp.sum(jnp.where(rk == r_row, ival, 0.0),
                            axis=0, keepdims=True)
    gidx_ref[0] = acc.astype(jnp.int32)


def _score_topk(h, sf):
    return pl.pallas_call(
        _score_topk_kernel,
        grid=(B,),
        in_specs=[
            pl.BlockSpec((1, N, D), lambda b: (b, 0, 0)),
            pl.BlockSpec((1, D, 1), lambda b: (b, 0, 0)),
        ],
        out_specs=[
            pl.BlockSpec((1, N, D), lambda b: (b, 0, 0)),
            pl.BlockSpec((1, 1, K), lambda b: (b, 0, 0)),
        ],
        out_shape=[
            jax.ShapeDtypeStruct((B, N, D), jnp.float32),
            jax.ShapeDtypeStruct((B, 1, K), jnp.int32),
        ],
    )(h, jnp.transpose(sf, (0, 2, 1)))


NC = 2
NS = 16
NW = NC * NS
ROWS_PER_W = (B * K) // NW
GCHUNK = 16


def _gather_kernel(g1_hbm, hs_hbm, idx_hbm, outg_hbm, outh_hbm,
                   idx_v, gbuf, hbuf, gsem, hsem):
    wid = lax.axis_index("s") * NC + lax.axis_index("c")
    base = wid * ROWS_PER_W
    nch = ROWS_PER_W // GCHUNK
    for k in range(nch):
        pltpu.sync_copy(idx_hbm.at[pl.ds(base + k * GCHUNK, GCHUNK)],
                        idx_v[k])
    for k in range(nch):
        cg = pltpu.async_copy(g1_hbm.at[idx_v[k]], gbuf[k % 2], gsem)
        ch = pltpu.async_copy(hs_hbm.at[idx_v[k]], hbuf[k % 2], hsem)
        cg.wait()
        ch.wait()
        pltpu.sync_copy(gbuf[k % 2],
                        outg_hbm.at[pl.ds(base + k * GCHUNK, GCHUNK)])
        pltpu.sync_copy(hbuf[k % 2],
                        outh_hbm.at[pl.ds(base + k * GCHUNK, GCHUNK)])


def _gather(g1_flat, hs_flat, gidx):
    mesh = plsc.VectorSubcoreMesh(core_axis_name="c", subcore_axis_name="s")
    nch = ROWS_PER_W // GCHUNK
    fn = functools.partial(
        pl.kernel,
        mesh=mesh,
        out_type=[
            jax.ShapeDtypeStruct((B * K, N), jnp.float32),
            jax.ShapeDtypeStruct((B * K, D), jnp.float32),
        ],
        scratch_types=[
            [pltpu.VMEM((GCHUNK,), jnp.int32) for _ in range(nch)],
            [pltpu.VMEM((GCHUNK, N), jnp.float32) for _ in range(2)],
            [pltpu.VMEM((GCHUNK, D), jnp.float32) for _ in range(2)],
            pltpu.SemaphoreType.DMA,
            pltpu.SemaphoreType.DMA,
        ],
    )(_gather_kernel)
    return fn(g1_flat, hs_flat, gidx)


def kernel(g1, h, section_feature):
    hs, gidx = _score_topk(h, section_feature)
    g1_flat = g1.reshape(B * N, N)
    hs_flat = hs.reshape(B * N, D)
    new_g, new_h = _gather(g1_flat, hs_flat, gidx.reshape(B * K))
    return new_g.reshape(B, K, N), new_h.reshape(B, K, D)


# speedup vs baseline: 1.1997x; 1.0209x over previous
import functools

import jax
import jax.numpy as jnp
from jax import lax
from jax.experimental import pallas as pl
from jax.experimental.pallas import tpu as pltpu
from jax.experimental.pallas import tpu_sc as plsc

N = 2048
D = 128
B = 4
K = 512
CH = 256


def _score_topk_kernel(h_ref, sf_ref, hs_ref, gidx_ref):
    b = pl.program_id(0)
    h = h_ref[0]
    sf = sf_ref[0]
    w_col = lax.dot_general(
        h.astype(jnp.bfloat16), sf.astype(jnp.bfloat16),
        (((1,), (0,)), ((), ())),
        preferred_element_type=jnp.float32)
    s_col = jax.nn.sigmoid(w_col)
    hs_ref[0] = h * s_col
    s_row = jnp.transpose(s_col)

    CB = 512
    nch = N // CB
    tri = (lax.broadcasted_iota(jnp.int32, (CB, CB), 1) <
           lax.broadcasted_iota(jnp.int32, (CB, CB), 0))
    col_parts = []
    row_parts = [jnp.zeros((1, CB), jnp.float32) for _ in range(nch)]
    for a in range(nch):
        sa_col = lax.slice(s_col, (a * CB, 0), ((a + 1) * CB, 1))
        sa_row = lax.slice(s_row, (0, a * CB), (1, (a + 1) * CB))
        g = (sa_row > sa_col) | ((sa_row == sa_col) & tri)
        acc_a = jnp.sum(g.astype(jnp.float32), axis=1, keepdims=True)
        for bb in range(a + 1, nch):
            sb_row = lax.slice(s_row, (0, bb * CB), (1, (bb + 1) * CB))
            G = (sb_row > sa_col).astype(jnp.float32)
            acc_a = acc_a + jnp.sum(G, axis=1, keepdims=True)
            row_parts[bb] = row_parts[bb] + (
                CB - jnp.sum(G, axis=0, keepdims=True))
        col_parts.append(acc_a)
    rank_col = (jnp.concatenate(col_parts, axis=0) +
                jnp.transpose(jnp.concatenate(row_parts, axis=1)))

    r_row = lax.broadcasted_iota(jnp.int32, (1, K), 1).astype(jnp.float32)
    acc = jnp.zeros((1, K), jnp.float32)
    for c in range(N // CH):
        rk = lax.slice(rank_col, (c * CH, 0), ((c + 1) * CH, 1))
        ival = (b * N + c * CH).astype(jnp.float32) + \
            lax.broadcasted_iota(jnp.int32, (CH, 1), 0).astype(jnp.float32)
        acc = acc + jnp.sum(jnp.where(rk == r_row, ival, 0.0),
                            axis=0, keepdims=True)
    gidx_ref[0] = acc.astype(jnp.int32)


def _score_topk(h, sf):
    return pl.pallas_call(
        _score_topk_kernel,
        grid=(B,),
        in_specs=[
            pl.BlockSpec((1, N, D), lambda b: (b, 0, 0)),
            pl.BlockSpec((1, D, 1), lambda b: (b, 0, 0)),
        ],
        out_specs=[
            pl.BlockSpec((1, N, D), lambda b: (b, 0, 0)),
            pl.BlockSpec((1, 1, K), lambda b: (b, 0, 0)),
        ],
        out_shape=[
            jax.ShapeDtypeStruct((B, N, D), jnp.float32),
            jax.ShapeDtypeStruct((B, 1, K), jnp.int32),
        ],
    )(h, jnp.transpose(sf, (0, 2, 1)))


NC = 2
NS = 16
NW = NC * NS
ROWS_PER_W = (B * K) // NW
GCHUNK = 16


def _gather_kernel(g1_hbm, hs_hbm, idx_hbm, outg_hbm, outh_hbm,
                   idx_v, gbuf, hbuf, gsem, hsem):
    wid = lax.axis_index("s") * NC + lax.axis_index("c")
    base = wid * ROWS_PER_W
    nch = ROWS_PER_W // GCHUNK
    for k in range(nch):
        pltpu.sync_copy(idx_hbm.at[pl.ds(base + k * GCHUNK, GCHUNK)],
                        idx_v[k])
    for k in range(nch):
        cg = pltpu.async_copy(g1_hbm.at[idx_v[k]], gbuf[k % 2], gsem)
        ch = pltpu.async_copy(hs_hbm.at[idx_v[k]], hbuf[k % 2], hsem)
        cg.wait()
        ch.wait()
        pltpu.sync_copy(gbuf[k % 2],
                        outg_hbm.at[pl.ds(base + k * GCHUNK, GCHUNK)])
        pltpu.sync_copy(hbuf[k % 2],
                        outh_hbm.at[pl.ds(base + k * GCHUNK, GCHUNK)])


def _gather(g1_flat, hs_flat, gidx):
    mesh = plsc.VectorSubcoreMesh(core_axis_name="c", subcore_axis_name="s")
    nch = ROWS_PER_W // GCHUNK
    fn = functools.partial(
        pl.kernel,
        mesh=mesh,
        out_type=[
            jax.ShapeDtypeStruct((B * K, N), jnp.float32),
            jax.ShapeDtypeStruct((B * K, D), jnp.float32),
        ],
        scratch_types=[
            [pltpu.VMEM((GCHUNK,), jnp.int32) for _ in range(nch)],
            [pltpu.VMEM((GCHUNK, N), jnp.float32) for _ in range(2)],
            [pltpu.VMEM((GCHUNK, D), jnp.float32) for _ in range(2)],
            pltpu.SemaphoreType.DMA,
            pltpu.SemaphoreType.DMA,
        ],
    )(_gather_kernel)
    return fn(g1_flat, hs_flat, gidx)


def kernel(g1, h, section_feature):
    hs, gidx = _score_topk(h, section_feature)
    g1_flat = g1.reshape(B * N, N)
    hs_flat = hs.reshape(B * N, D)
    new_g, new_h = _gather(g1_flat, hs_flat, gidx.reshape(B * K))
    return new_g.reshape(B, K, N), new_h.reshape(B, K, D)


# speedup vs baseline: 1.2632x; 1.0530x over previous
import functools

import jax
import jax.numpy as jnp
from jax import lax
from jax.experimental import pallas as pl
from jax.experimental.pallas import tpu as pltpu
from jax.experimental.pallas import tpu_sc as plsc

N = 2048
D = 128
B = 4
K = 512
CH = 256


def _score_topk_kernel(h_ref, sf_ref, hs_ref, gidx_ref):
    b = pl.program_id(0)
    h = h_ref[0]
    sf = jnp.transpose(sf_ref[0])
    w_col = lax.dot_general(
        h.astype(jnp.bfloat16), sf.astype(jnp.bfloat16),
        (((1,), (0,)), ((), ())),
        preferred_element_type=jnp.float32)
    s_col = jax.nn.sigmoid(w_col)
    hs_ref[0] = h * s_col
    s_row = jnp.transpose(s_col)

    CB = 256
    nch = N // CB
    tri = (lax.broadcasted_iota(jnp.int32, (CB, CB), 1) <
           lax.broadcasted_iota(jnp.int32, (CB, CB), 0))
    col_parts = []
    row_parts = [jnp.zeros((1, CB), jnp.float32) for _ in range(nch)]
    for a in range(nch):
        sa_col = lax.slice(s_col, (a * CB, 0), ((a + 1) * CB, 1))
        sa_row = lax.slice(s_row, (0, a * CB), (1, (a + 1) * CB))
        g = (sa_row > sa_col) | ((sa_row == sa_col) & tri)
        acc_a = jnp.sum(g.astype(jnp.float32), axis=1, keepdims=True)
        for bb in range(a + 1, nch):
            sb_row = lax.slice(s_row, (0, bb * CB), (1, (bb + 1) * CB))
            G = (sb_row > sa_col).astype(jnp.float32)
            acc_a = acc_a + jnp.sum(G, axis=1, keepdims=True)
            row_parts[bb] = row_parts[bb] + (
                CB - jnp.sum(G, axis=0, keepdims=True))
        col_parts.append(acc_a)
    rank_col = (jnp.concatenate(col_parts, axis=0) +
                jnp.transpose(jnp.concatenate(row_parts, axis=1)))

    r_row = lax.broadcasted_iota(jnp.int32, (1, K), 1).astype(jnp.float32)
    acc = jnp.zeros((1, K), jnp.float32)
    for c in range(N // CH):
        rk = lax.slice(rank_col, (c * CH, 0), ((c + 1) * CH, 1))
        ival = (b * N + c * CH).astype(jnp.float32) + \
            lax.broadcasted_iota(jnp.int32, (CH, 1), 0).astype(jnp.float32)
        acc = acc + jnp.sum(jnp.where(rk == r_row, ival, 0.0),
                            axis=0, keepdims=True)
    gidx_ref[0] = acc.astype(jnp.int32)


def _score_topk(h, sf):
    return pl.pallas_call(
        _score_topk_kernel,
        grid=(B,),
        in_specs=[
            pl.BlockSpec((1, N, D), lambda b: (b, 0, 0)),
            pl.BlockSpec((1, 1, D), lambda b: (b, 0, 0)),
        ],
        out_specs=[
            pl.BlockSpec((1, N, D), lambda b: (b, 0, 0)),
            pl.BlockSpec((1, 1, K), lambda b: (b, 0, 0)),
        ],
        out_shape=[
            jax.ShapeDtypeStruct((B, N, D), jnp.float32),
            jax.ShapeDtypeStruct((B, 1, K), jnp.int32),
        ],
    )(h, sf)


NC = 2
NS = 16
NW = NC * NS
ROWS_PER_W = (B * K) // NW
GCHUNK = 16


def _gather_kernel(g1_hbm, hs_hbm, idx_hbm, outg_hbm, outh_hbm,
                   idx_v, gbuf, hbuf, gsem, hsem, wgsem, whsem):
    wid = lax.axis_index("s") * NC + lax.axis_index("c")
    base = wid * ROWS_PER_W
    nch = ROWS_PER_W // GCHUNK
    pltpu.sync_copy(idx_hbm.at[pl.ds(base, ROWS_PER_W)], idx_v)
    gc = [None] * nch
    hc = [None] * nch
    wg = [None] * nch
    wh = [None] * nch

    def start_gather(k):
        sl = idx_v.at[pl.ds(k * GCHUNK, GCHUNK)]
        gc[k] = pltpu.async_copy(g1_hbm.at[sl], gbuf[k % 2], gsem)
        hc[k] = pltpu.async_copy(hs_hbm.at[sl], hbuf[k % 2], hsem)

    start_gather(0)
    for k in range(nch):
        if k + 1 < nch:
            if k - 1 >= 0:
                wg[k - 1].wait()
                wh[k - 1].wait()
            start_gather(k + 1)
        gc[k].wait()
        hc[k].wait()
        dst = pl.ds(base + k * GCHUNK, GCHUNK)
        wg[k] = pltpu.async_copy(gbuf[k % 2], outg_hbm.at[dst], wgsem)
        wh[k] = pltpu.async_copy(hbuf[k % 2], outh_hbm.at[dst], whsem)
    for k in range(max(0, nch - 2), nch):
        wg[k].wait()
        wh[k].wait()


def _gather(g1_flat, hs_flat, gidx):
    mesh = plsc.VectorSubcoreMesh(core_axis_name="c", subcore_axis_name="s")
    fn = functools.partial(
        pl.kernel,
        mesh=mesh,
        out_type=[
            jax.ShapeDtypeStruct((B * K, N), jnp.float32),
            jax.ShapeDtypeStruct((B * K, D), jnp.float32),
        ],
        scratch_types=[
            pltpu.VMEM((ROWS_PER_W,), jnp.int32),
            [pltpu.VMEM((GCHUNK, N), jnp.float32) for _ in range(2)],
            [pltpu.VMEM((GCHUNK, D), jnp.float32) for _ in range(2)],
            pltpu.SemaphoreType.DMA,
            pltpu.SemaphoreType.DMA,
            pltpu.SemaphoreType.DMA,
            pltpu.SemaphoreType.DMA,
        ],
    )(_gather_kernel)
    return fn(g1_flat, hs_flat, gidx)


def kernel(g1, h, section_feature):
    hs, gidx = _score_topk(h, section_feature)
    g1_flat = g1.reshape(B * N, N)
    hs_flat = hs.reshape(B * N, D)
    new_g, new_h = _gather(g1_flat, hs_flat, gidx.reshape(B * K))
    return new_g.reshape(B, K, N), new_h.reshape(B, K, D)


# speedup vs baseline: 1.3253x; 1.0491x over previous
import functools

import jax
import jax.numpy as jnp
from jax import lax
from jax.experimental import pallas as pl
from jax.experimental.pallas import tpu as pltpu
from jax.experimental.pallas import tpu_sc as plsc

N = 2048
D = 128
B = 4
K = 512
CH = 256


def _score_topk_kernel(h_ref, sf_ref, hs_ref, gidx_ref):
    b = pl.program_id(0)
    h = h_ref[0]
    sf = jnp.transpose(sf_ref[0])
    w_col = lax.dot_general(
        h.astype(jnp.bfloat16), sf.astype(jnp.bfloat16),
        (((1,), (0,)), ((), ())),
        preferred_element_type=jnp.float32)
    s_col = jax.nn.sigmoid(w_col)
    hs_ref[0] = h * s_col
    s_row = jnp.transpose(s_col)

    CB = 256
    nch = N // CB
    tri = (lax.broadcasted_iota(jnp.int32, (CB, CB), 1) <
           lax.broadcasted_iota(jnp.int32, (CB, CB), 0))
    ones_col = jnp.ones((CB, 1), jnp.float32)
    ones_row = jnp.ones((1, CB), jnp.float32)
    dn_row = (((1,), (0,)), ((), ()))
    dn_col = (((1,), (0,)), ((), ()))
    col_parts = []
    row_parts = [jnp.zeros((1, CB), jnp.float32) for _ in range(nch)]
    for a in range(nch):
        sa_col = lax.slice(s_col, (a * CB, 0), ((a + 1) * CB, 1))
        sa_row = lax.slice(s_row, (0, a * CB), (1, (a + 1) * CB))
        g = ((sa_row > sa_col) | ((sa_row == sa_col) & tri)).astype(
            jnp.float32)
        acc_a = lax.dot_general(g, ones_col, dn_row,
                                preferred_element_type=jnp.float32)
        for bb in range(a + 1, nch):
            sb_row = lax.slice(s_row, (0, bb * CB), (1, (bb + 1) * CB))
            G = (sb_row > sa_col).astype(jnp.float32)
            acc_a = acc_a + lax.dot_general(
                G, ones_col, dn_row, preferred_element_type=jnp.float32)
            row_parts[bb] = row_parts[bb] + (
                CB - lax.dot_general(ones_row, G, dn_col,
                                     preferred_element_type=jnp.float32))
        col_parts.append(acc_a)
    rank_col = (jnp.concatenate(col_parts, axis=0) +
                jnp.transpose(jnp.concatenate(row_parts, axis=1)))

    r_row = lax.broadcasted_iota(jnp.int32, (1, K), 1).astype(jnp.float32)
    li_row = lax.broadcasted_iota(jnp.int32, (1, CH), 1).astype(jnp.float32)
    acc = jnp.zeros((1, K), jnp.float32)
    for c in range(N // CH):
        rk = lax.slice(rank_col, (c * CH, 0), ((c + 1) * CH, 1))
        onehot = (rk == r_row).astype(jnp.float32)
        cnt = lax.dot_general(ones_row, onehot, dn_col,
                              preferred_element_type=jnp.float32)
        lid = lax.dot_general(li_row, onehot, dn_col,
                              preferred_element_type=jnp.float32)
        base = (b * N + c * CH).astype(jnp.float32)
        acc = acc + base * cnt + lid
    gidx_ref[0] = acc.astype(jnp.int32)


def _score_topk(h, sf):
    return pl.pallas_call(
        _score_topk_kernel,
        grid=(B,),
        in_specs=[
            pl.BlockSpec((1, N, D), lambda b: (b, 0, 0)),
            pl.BlockSpec((1, 1, D), lambda b: (b, 0, 0)),
        ],
        out_specs=[
            pl.BlockSpec((1, N, D), lambda b: (b, 0, 0)),
            pl.BlockSpec((1, 1, K), lambda b: (b, 0, 0)),
        ],
        out_shape=[
            jax.ShapeDtypeStruct((B, N, D), jnp.float32),
            jax.ShapeDtypeStruct((B, 1, K), jnp.int32),
        ],
    )(h, sf)


NC = 2
NS = 16
NW = NC * NS
ROWS_PER_W = (B * K) // NW
GCHUNK = 16


def _gather_kernel(g1_hbm, hs_hbm, idx_hbm, outg_hbm, outh_hbm,
                   idx_v, gbuf, hbuf, gsem, hsem, wgsem, whsem):
    wid = lax.axis_index("s") * NC + lax.axis_index("c")
    base = wid * ROWS_PER_W
    nch = ROWS_PER_W // GCHUNK
    pltpu.sync_copy(idx_hbm.at[pl.ds(base, ROWS_PER_W)], idx_v)
    gc = [None] * nch
    hc = [None] * nch
    wg = [None] * nch
    wh = [None] * nch

    def start_gather(k):
        sl = idx_v.at[pl.ds(k * GCHUNK, GCHUNK)]
        gc[k] = pltpu.async_copy(g1_hbm.at[sl], gbuf[k % 2], gsem)
        hc[k] = pltpu.async_copy(hs_hbm.at[sl], hbuf[k % 2], hsem)

    start_gather(0)
    for k in range(nch):
        if k + 1 < nch:
            if k - 1 >= 0:
                wg[k - 1].wait()
                wh[k - 1].wait()
            start_gather(k + 1)
        gc[k].wait()
        hc[k].wait()
        dst = pl.ds(base + k * GCHUNK, GCHUNK)
        wg[k] = pltpu.async_copy(gbuf[k % 2], outg_hbm.at[dst], wgsem)
        wh[k] = pltpu.async_copy(hbuf[k % 2], outh_hbm.at[dst], whsem)
    for k in range(max(0, nch - 2), nch):
        wg[k].wait()
        wh[k].wait()


def _gather(g1_flat, hs_flat, gidx):
    mesh = plsc.VectorSubcoreMesh(core_axis_name="c", subcore_axis_name="s")
    fn = functools.partial(
        pl.kernel,
        mesh=mesh,
        out_type=[
            jax.ShapeDtypeStruct((B * K, N), jnp.float32),
            jax.ShapeDtypeStruct((B * K, D), jnp.float32),
        ],
        scratch_types=[
            pltpu.VMEM((ROWS_PER_W,), jnp.int32),
            [pltpu.VMEM((GCHUNK, N), jnp.float32) for _ in range(2)],
            [pltpu.VMEM((GCHUNK, D), jnp.float32) for _ in range(2)],
            pltpu.SemaphoreType.DMA,
            pltpu.SemaphoreType.DMA,
            pltpu.SemaphoreType.DMA,
            pltpu.SemaphoreType.DMA,
        ],
    )(_gather_kernel)
    return fn(g1_flat, hs_flat, gidx)


def kernel(g1, h, section_feature):
    hs, gidx = _score_topk(h, section_feature)
    g1_flat = g1.reshape(B * N, N)
    hs_flat = hs.reshape(B * N, D)
    new_g, new_h = _gather(g1_flat, hs_flat, gidx.reshape(B * K))
    return new_g.reshape(B, K, N), new_h.reshape(B, K, D)


# speedup vs baseline: 1.3257x; 1.0003x over previous
import functools

import jax
import jax.numpy as jnp
from jax import lax
from jax.experimental import pallas as pl
from jax.experimental.pallas import tpu as pltpu
from jax.experimental.pallas import tpu_sc as plsc

N = 2048
D = 128
B = 4
K = 512
CH = 256


def _score_topk_kernel(h_ref, sf_ref, hs_ref, gidx_ref):
    b = pl.program_id(0)
    h = h_ref[0]
    sf = jnp.transpose(sf_ref[0])
    w_col = lax.dot_general(
        h.astype(jnp.bfloat16), sf.astype(jnp.bfloat16),
        (((1,), (0,)), ((), ())),
        preferred_element_type=jnp.float32)
    s_col = jax.nn.sigmoid(w_col)
    hs_ref[0] = h * s_col
    s_row = jnp.transpose(s_col)

    CB = 256
    nch = N // CB
    tri = (lax.broadcasted_iota(jnp.int32, (CB, CB), 1) <
           lax.broadcasted_iota(jnp.int32, (CB, CB), 0))
    ones_col = jnp.ones((CB, 1), jnp.float32)
    ones_row = jnp.ones((1, CB), jnp.float32)
    dn_row = (((1,), (0,)), ((), ()))
    dn_col = (((1,), (0,)), ((), ()))
    col_parts = []
    row_parts = [jnp.zeros((1, CB), jnp.float32) for _ in range(nch)]
    for a in range(nch):
        sa_col = lax.slice(s_col, (a * CB, 0), ((a + 1) * CB, 1))
        sa_row = lax.slice(s_row, (0, a * CB), (1, (a + 1) * CB))
        g = ((sa_row > sa_col) | ((sa_row == sa_col) & tri)).astype(
            jnp.float32)
        acc_a = lax.dot_general(g, ones_col, dn_row,
                                preferred_element_type=jnp.float32)
        for bb in range(a + 1, nch):
            sb_row = lax.slice(s_row, (0, bb * CB), (1, (bb + 1) * CB))
            G = (sb_row > sa_col).astype(jnp.float32)
            acc_a = acc_a + lax.dot_general(
                G, ones_col, dn_row, preferred_element_type=jnp.float32)
            row_parts[bb] = row_parts[bb] + (
                CB - lax.dot_general(ones_row, G, dn_col,
                                     preferred_element_type=jnp.float32))
        col_parts.append(acc_a)
    rank_col = (jnp.concatenate(col_parts, axis=0) +
                jnp.transpose(jnp.concatenate(row_parts, axis=1)))

    r_row = lax.broadcasted_iota(jnp.int32, (1, K), 1).astype(jnp.float32)
    li_row = lax.broadcasted_iota(jnp.int32, (1, CH), 1).astype(jnp.float32)
    acc = jnp.zeros((1, K), jnp.float32)
    for c in range(N // CH):
        rk = lax.slice(rank_col, (c * CH, 0), ((c + 1) * CH, 1))
        onehot = (rk == r_row).astype(jnp.float32)
        cnt = lax.dot_general(ones_row, onehot, dn_col,
                              preferred_element_type=jnp.float32)
        lid = lax.dot_general(li_row, onehot, dn_col,
                              preferred_element_type=jnp.float32)
        base = (b * N + c * CH).astype(jnp.float32)
        acc = acc + base * cnt + lid
    gidx_ref[0] = acc.astype(jnp.int32)


def _score_topk(h, sf):
    return pl.pallas_call(
        _score_topk_kernel,
        grid=(B,),
        in_specs=[
            pl.BlockSpec((1, N, D), lambda b: (b, 0, 0)),
            pl.BlockSpec((1, 1, D), lambda b: (b, 0, 0)),
        ],
        out_specs=[
            pl.BlockSpec((1, N, D), lambda b: (b, 0, 0)),
            pl.BlockSpec((1, 1, K), lambda b: (b, 0, 0)),
        ],
        out_shape=[
            jax.ShapeDtypeStruct((B, N, D), jnp.float32),
            jax.ShapeDtypeStruct((B, 1, K), jnp.int32),
        ],
    )(h, sf)


NC = 2
NS = 16
NW = NC * NS
ROWS_PER_W = (B * K) // NW
GCHUNK = 8
NBUF = 4


def _gather_kernel(g1_hbm, hs_hbm, idx_hbm, outg_hbm, outh_hbm,
                   idx_v, gbuf, hbuf, gsem, hsem, wgsem, whsem):
    wid = lax.axis_index("s") * NC + lax.axis_index("c")
    base = wid * ROWS_PER_W
    nch = ROWS_PER_W // GCHUNK
    pltpu.sync_copy(idx_hbm.at[pl.ds(base, ROWS_PER_W)], idx_v)
    gc = [None] * nch
    hc = [None] * nch
    wg = [None] * nch
    wh = [None] * nch

    def start_gather(k):
        sl = idx_v.at[pl.ds(k * GCHUNK, GCHUNK)]
        gc[k] = pltpu.async_copy(g1_hbm.at[sl], gbuf[k % NBUF], gsem)
        hc[k] = pltpu.async_copy(hs_hbm.at[sl], hbuf[k % NBUF], hsem)

    for k in range(NBUF):
        start_gather(k)
    waited = set()
    for k in range(nch):
        gc[k].wait()
        hc[k].wait()
        dst = pl.ds(base + k * GCHUNK, GCHUNK)
        wg[k] = pltpu.async_copy(gbuf[k % NBUF], outg_hbm.at[dst], wgsem)
        wh[k] = pltpu.async_copy(hbuf[k % NBUF], outh_hbm.at[dst], whsem)
        if k + NBUF < nch:
            wg[k].wait()
            wh[k].wait()
            waited.add(k)
            start_gather(k + NBUF)
    for k in range(nch):
        if k not in waited:
            wg[k].wait()
            wh[k].wait()


def _gather(g1_flat, hs_flat, gidx):
    mesh = plsc.VectorSubcoreMesh(core_axis_name="c", subcore_axis_name="s")
    fn = functools.partial(
        pl.kernel,
        mesh=mesh,
        out_type=[
            jax.ShapeDtypeStruct((B * K, N), jnp.float32),
            jax.ShapeDtypeStruct((B * K, D), jnp.float32),
        ],
        scratch_types=[
            pltpu.VMEM((ROWS_PER_W,), jnp.int32),
            [pltpu.VMEM((GCHUNK, N), jnp.float32) for _ in range(NBUF)],
            [pltpu.VMEM((GCHUNK, D), jnp.float32) for _ in range(NBUF)],
            pltpu.SemaphoreType.DMA,
            pltpu.SemaphoreType.DMA,
            pltpu.SemaphoreType.DMA,
            pltpu.SemaphoreType.DMA,
        ],
    )(_gather_kernel)
    return fn(g1_flat, hs_flat, gidx)


def kernel(g1, h, section_feature):
    hs, gidx = _score_topk(h, section_feature)
    g1_flat = g1.reshape(B * N, N)
    hs_flat = hs.reshape(B * N, D)
    new_g, new_h = _gather(g1_flat, hs_flat, gidx.reshape(B * K))
    return new_g.reshape(B, K, N), new_h.reshape(B, K, D)


# speedup vs baseline: 1.3420x; 1.0123x over previous
import functools

import jax
import jax.numpy as jnp
from jax import lax
from jax.experimental import pallas as pl
from jax.experimental.pallas import tpu as pltpu
from jax.experimental.pallas import tpu_sc as plsc

N = 2048
D = 128
B = 4
K = 512
CH = 256


def _score_topk_kernel(h_ref, sf_ref, hs_ref, gidx_ref):
    b = pl.program_id(0)
    h = h_ref[0]
    sf = jnp.transpose(sf_ref[0])
    w_col = lax.dot_general(
        h.astype(jnp.bfloat16), sf.astype(jnp.bfloat16),
        (((1,), (0,)), ((), ())),
        preferred_element_type=jnp.float32)
    s_col = jax.nn.sigmoid(w_col)
    hs_ref[0] = h * s_col
    s_row = jnp.transpose(s_col)

    CB = 256
    nch = N // CB
    tri = (lax.broadcasted_iota(jnp.int32, (CB, CB), 1) <
           lax.broadcasted_iota(jnp.int32, (CB, CB), 0))
    ones_col = jnp.ones((CB, 1), jnp.bfloat16)
    ones_row = jnp.ones((1, CB), jnp.bfloat16)
    dn_row = (((1,), (0,)), ((), ()))
    dn_col = (((1,), (0,)), ((), ()))
    col_parts = []
    row_parts = [jnp.zeros((1, CB), jnp.float32) for _ in range(nch)]
    for a in range(nch):
        sa_col = lax.slice(s_col, (a * CB, 0), ((a + 1) * CB, 1))
        sa_row = lax.slice(s_row, (0, a * CB), (1, (a + 1) * CB))
        g = ((sa_row > sa_col) | ((sa_row == sa_col) & tri)).astype(
            jnp.bfloat16)
        acc_a = lax.dot_general(g, ones_col, dn_row,
                                preferred_element_type=jnp.float32)
        for bb in range(a + 1, nch):
            sb_row = lax.slice(s_row, (0, bb * CB), (1, (bb + 1) * CB))
            G = (sb_row > sa_col).astype(jnp.bfloat16)
            acc_a = acc_a + lax.dot_general(
                G, ones_col, dn_row, preferred_element_type=jnp.float32)
            row_parts[bb] = row_parts[bb] + (
                CB - lax.dot_general(ones_row, G, dn_col,
                                     preferred_element_type=jnp.float32))
        col_parts.append(acc_a)
    rank_col = (jnp.concatenate(col_parts, axis=0) +
                jnp.transpose(jnp.concatenate(row_parts, axis=1)))

    r_row = lax.broadcasted_iota(jnp.int32, (1, K), 1).astype(jnp.float32)
    li_row = lax.broadcasted_iota(jnp.int32, (1, CH), 1).astype(jnp.bfloat16)
    acc = jnp.zeros((1, K), jnp.float32)
    for c in range(N // CH):
        rk = lax.slice(rank_col, (c * CH, 0), ((c + 1) * CH, 1))
        onehot = (rk == r_row).astype(jnp.bfloat16)
        cnt = lax.dot_general(ones_row, onehot, dn_col,
                              preferred_element_type=jnp.float32)
        lid = lax.dot_general(li_row, onehot, dn_col,
                              preferred_element_type=jnp.float32)
        base = (b * N + c * CH).astype(jnp.float32)
        acc = acc + base * cnt + lid
    gidx_ref[0] = acc.astype(jnp.int32)


def _score_topk(h, sf):
    return pl.pallas_call(
        _score_topk_kernel,
        grid=(B,),
        in_specs=[
            pl.BlockSpec((1, N, D), lambda b: (b, 0, 0)),
            pl.BlockSpec((1, 1, D), lambda b: (b, 0, 0)),
        ],
        out_specs=[
            pl.BlockSpec((1, N, D), lambda b: (b, 0, 0)),
            pl.BlockSpec((1, 1, K), lambda b: (b, 0, 0)),
        ],
        out_shape=[
            jax.ShapeDtypeStruct((B, N, D), jnp.float32),
            jax.ShapeDtypeStruct((B, 1, K), jnp.int32),
        ],
    )(h, sf)


NC = 2
NS = 16
NW = NC * NS
ROWS_PER_W = (B * K) // NW
GCHUNK = 8
NBUF = 4


def _gather_kernel(g1_hbm, hs_hbm, idx_hbm, outg_hbm, outh_hbm,
                   idx_v, gbuf, hbuf, gsem, hsem, wgsem, whsem):
    wid = lax.axis_index("s") * NC + lax.axis_index("c")
    base = wid * ROWS_PER_W
    nch = ROWS_PER_W // GCHUNK
    pltpu.sync_copy(idx_hbm.at[pl.ds(base, ROWS_PER_W)], idx_v)
    gc = [None] * nch
    hc = [None] * nch
    wg = [None] * nch
    wh = [None] * nch

    def start_gather(k):
        sl = idx_v.at[pl.ds(k * GCHUNK, GCHUNK)]
        gc[k] = pltpu.async_copy(g1_hbm.at[sl], gbuf[k % NBUF], gsem)
        hc[k] = pltpu.async_copy(hs_hbm.at[sl], hbuf[k % NBUF], hsem)

    for k in range(NBUF):
        start_gather(k)
    waited = set()
    for k in range(nch):
        gc[k].wait()
        hc[k].wait()
        dst = pl.ds(base + k * GCHUNK, GCHUNK)
        wg[k] = pltpu.async_copy(gbuf[k % NBUF], outg_hbm.at[dst], wgsem)
        wh[k] = pltpu.async_copy(hbuf[k % NBUF], outh_hbm.at[dst], whsem)
        if k + NBUF < nch:
            wg[k].wait()
            wh[k].wait()
            waited.add(k)
            start_gather(k + NBUF)
    for k in range(nch):
        if k not in waited:
            wg[k].wait()
            wh[k].wait()


def _gather(g1_flat, hs_flat, gidx):
    mesh = plsc.VectorSubcoreMesh(core_axis_name="c", subcore_axis_name="s")
    fn = functools.partial(
        pl.kernel,
        mesh=mesh,
        out_type=[
            jax.ShapeDtypeStruct((B * K, N), jnp.float32),
            jax.ShapeDtypeStruct((B * K, D), jnp.float32),
        ],
        scratch_types=[
            pltpu.VMEM((ROWS_PER_W,), jnp.int32),
            [pltpu.VMEM((GCHUNK, N), jnp.float32) for _ in range(NBUF)],
            [pltpu.VMEM((GCHUNK, D), jnp.float32) for _ in range(NBUF)],
            pltpu.SemaphoreType.DMA,
            pltpu.SemaphoreType.DMA,
            pltpu.SemaphoreType.DMA,
            pltpu.SemaphoreType.DMA,
        ],
    )(_gather_kernel)
    return fn(g1_flat, hs_flat, gidx)


def kernel(g1, h, section_feature):
    hs, gidx = _score_topk(h, section_feature)
    g1_flat = g1.reshape(B * N, N)
    hs_flat = hs.reshape(B * N, D)
    new_g, new_h = _gather(g1_flat, hs_flat, gidx.reshape(B * K))
    return new_g.reshape(B, K, N), new_h.reshape(B, K, D)


# speedup vs baseline: 1.3477x; 1.0043x over previous
import functools

import jax
import jax.numpy as jnp
from jax import lax
from jax.experimental import pallas as pl
from jax.experimental.pallas import tpu as pltpu
from jax.experimental.pallas import tpu_sc as plsc

N = 2048
D = 128
B = 4
K = 512
CH = 256


def _score_topk_kernel(h_ref, sf_ref, hs_ref, gidx_ref):
    b = pl.program_id(0)
    h = h_ref[0]
    sf = jnp.transpose(sf_ref[0])
    w_col = lax.dot_general(
        h.astype(jnp.bfloat16), sf.astype(jnp.bfloat16),
        (((1,), (0,)), ((), ())),
        preferred_element_type=jnp.float32)
    s_col = jax.nn.sigmoid(w_col)
    hs_ref[0] = h * s_col
    s_row = jnp.transpose(s_col)

    CB = 256
    nch = N // CB
    tri = (lax.broadcasted_iota(jnp.int32, (CB, CB), 1) <
           lax.broadcasted_iota(jnp.int32, (CB, CB), 0))
    ones_col = jnp.ones((CB, 1), jnp.bfloat16)
    ones_row = jnp.ones((1, CB), jnp.bfloat16)
    dn_row = (((1,), (0,)), ((), ()))
    dn_col = (((1,), (0,)), ((), ()))
    col_parts = []
    row_parts = [jnp.zeros((1, CB), jnp.float32) for _ in range(nch)]
    for a in range(nch):
        sa_col = lax.slice(s_col, (a * CB, 0), ((a + 1) * CB, 1))
        sa_row = lax.slice(s_row, (0, a * CB), (1, (a + 1) * CB))
        g = ((sa_row > sa_col) | ((sa_row == sa_col) & tri)).astype(
            jnp.bfloat16)
        acc_a = lax.dot_general(g, ones_col, dn_row,
                                preferred_element_type=jnp.float32)
        for bb in range(a + 1, nch):
            sb_row = lax.slice(s_row, (0, bb * CB), (1, (bb + 1) * CB))
            G = (sb_row > sa_col).astype(jnp.bfloat16)
            acc_a = acc_a + lax.dot_general(
                G, ones_col, dn_row, preferred_element_type=jnp.float32)
            row_parts[bb] = row_parts[bb] + (
                CB - lax.dot_general(ones_row, G, dn_col,
                                     preferred_element_type=jnp.float32))
        col_parts.append(acc_a)
    rank_col = (jnp.concatenate(col_parts, axis=0) +
                jnp.transpose(jnp.concatenate(row_parts, axis=1)))

    r_row = lax.broadcasted_iota(jnp.int32, (1, K), 1).astype(jnp.float32)
    li_row = lax.broadcasted_iota(jnp.int32, (1, CH), 1).astype(jnp.bfloat16)
    acc = jnp.zeros((1, K), jnp.float32)
    for c in range(N // CH):
        rk = lax.slice(rank_col, (c * CH, 0), ((c + 1) * CH, 1))
        onehot = (rk == r_row).astype(jnp.bfloat16)
        cnt = lax.dot_general(ones_row, onehot, dn_col,
                              preferred_element_type=jnp.float32)
        lid = lax.dot_general(li_row, onehot, dn_col,
                              preferred_element_type=jnp.float32)
        base = (b * N + c * CH).astype(jnp.float32)
        acc = acc + base * cnt + lid
    gidx_ref[0] = acc.astype(jnp.int32)


def _score_topk(h, sf):
    return pl.pallas_call(
        _score_topk_kernel,
        grid=(B,),
        in_specs=[
            pl.BlockSpec((1, N, D), lambda b: (b, 0, 0)),
            pl.BlockSpec((1, 1, D), lambda b: (b, 0, 0)),
        ],
        out_specs=[
            pl.BlockSpec((1, N, D), lambda b: (b, 0, 0)),
            pl.BlockSpec((1, 1, K), lambda b: (b, 0, 0)),
        ],
        out_shape=[
            jax.ShapeDtypeStruct((B, N, D), jnp.float32),
            jax.ShapeDtypeStruct((B, 1, K), jnp.int32),
        ],
    )(h, sf)


NC = 2
NS = 16
NW = NC * NS
ROWS_PER_W = (B * K) // NW
GCHUNK = 16
NBUF = 3


def _gather_kernel(g1_hbm, hs_hbm, idx_hbm, outg_hbm, outh_hbm,
                   idx_v, gbuf, hbuf, gsem, hsem, wgsem, whsem):
    wid = lax.axis_index("s") * NC + lax.axis_index("c")
    base = wid * ROWS_PER_W
    nch = ROWS_PER_W // GCHUNK
    pltpu.sync_copy(idx_hbm.at[pl.ds(base, ROWS_PER_W)], idx_v)
    gc = [None] * nch
    hc = [None] * nch
    wg = [None] * nch
    wh = [None] * nch

    def start_gather(k):
        sl = idx_v.at[pl.ds(k * GCHUNK, GCHUNK)]
        gc[k] = pltpu.async_copy(g1_hbm.at[sl], gbuf[k % NBUF], gsem)
        hc[k] = pltpu.async_copy(hs_hbm.at[sl], hbuf[k % NBUF], hsem)

    for k in range(NBUF):
        start_gather(k)
    waited = set()
    for k in range(nch):
        gc[k].wait()
        hc[k].wait()
        dst = pl.ds(base + k * GCHUNK, GCHUNK)
        wg[k] = pltpu.async_copy(gbuf[k % NBUF], outg_hbm.at[dst], wgsem)
        wh[k] = pltpu.async_copy(hbuf[k % NBUF], outh_hbm.at[dst], whsem)
        if k + NBUF < nch:
            wg[k].wait()
            wh[k].wait()
            waited.add(k)
            start_gather(k + NBUF)
    for k in range(nch):
        if k not in waited:
            wg[k].wait()
            wh[k].wait()


def _gather(g1_flat, hs_flat, gidx):
    mesh = plsc.VectorSubcoreMesh(core_axis_name="c", subcore_axis_name="s")
    fn = functools.partial(
        pl.kernel,
        mesh=mesh,
        out_type=[
            jax.ShapeDtypeStruct((B * K, N), jnp.float32),
            jax.ShapeDtypeStruct((B * K, D), jnp.float32),
        ],
        scratch_types=[
            pltpu.VMEM((ROWS_PER_W,), jnp.int32),
            [pltpu.VMEM((GCHUNK, N), jnp.float32) for _ in range(NBUF)],
            [pltpu.VMEM((GCHUNK, D), jnp.float32) for _ in range(NBUF)],
            pltpu.SemaphoreType.DMA,
            pltpu.SemaphoreType.DMA,
            pltpu.SemaphoreType.DMA,
            pltpu.SemaphoreType.DMA,
        ],
    )(_gather_kernel)
    return fn(g1_flat, hs_flat, gidx)


def kernel(g1, h, section_feature):
    hs, gidx = _score_topk(h, section_feature)
    g1_flat = g1.reshape(B * N, N)
    hs_flat = hs.reshape(B * N, D)
    new_g, new_h = _gather(g1_flat, hs_flat, gidx.reshape(B * K))
    return new_g.reshape(B, K, N), new_h.reshape(B, K, D)
